# baseline (speedup 1.0000x reference)
import functools
import math

import jax
import jax.numpy as jnp
from jax import lax
from jax.experimental import pallas as pl
from jax.experimental.pallas import tpu as pltpu


def _erf(z):
    a1, a2, a3, a4, a5 = (0.254829592, -0.284496736, 1.421413741,
                          -1.453152027, 1.061405429)
    p = 0.3275911
    s = jnp.sign(z)
    za = jnp.abs(z)
    t = 1.0 / (1.0 + p * za)
    poly = t * (a1 + t * (a2 + t * (a3 + t * (a4 + t * a5))))
    return s * (1.0 - poly * jnp.exp(-za * za))


def _gelu(x):
    return 0.5 * x * (1.0 + _erf(x * (1.0 / math.sqrt(2.0))))


def _router_body(nsteps, x_ref, w_ref, bsw_ref, gv_ref, gi_ref, acc_ref):
    g = pl.program_id(0)

    @pl.when(g == 0)
    def _init():
        acc_ref[...] = jnp.zeros_like(acc_ref)

    acc_ref[...] += jnp.dot(x_ref[...], w_ref[...],
                            preferred_element_type=jnp.float32)

    @pl.when(g == nsteps - 1)
    def _finish():
        logits = acc_ref[...] + bsw_ref[...]
        m = jnp.max(logits, axis=1, keepdims=True)
        ex = jnp.exp(logits - m)
        probs = ex / jnp.sum(ex, axis=1, keepdims=True)
        Bn, En = probs.shape
        idx = lax.broadcasted_iota(jnp.int32, (Bn, En), 1)
        m0 = jnp.max(probs, axis=1, keepdims=True)
        i0 = jnp.min(jnp.where(probs == m0, idx, En), axis=1, keepdims=True)
        masked = jnp.where(idx == i0, -1.0, probs)
        m1 = jnp.max(masked, axis=1, keepdims=True)
        i1 = jnp.min(jnp.where(masked == m1, idx, En), axis=1, keepdims=True)
        gv_ref[...] = jnp.where(idx == 0, m0, jnp.where(idx == 1, m1, 0.0))
        gi_ref[...] = jnp.where(idx == 0, i0, jnp.where(idx == 1, i1, 0))


def _moe_body(S, T, HD, G,
              x_ref, gv_ref, gi_ref, w1_ref, b1_ref, w2_ref, b2_ref,
              w3_ref, b3_ref, out_ref, wa_ref, wb_ref, b1c_ref, b2c_ref,
              y_ref):
    b = pl.program_id(0)
    s = pl.program_id(1)
    GW = G * HD
    HW = 2 * GW

    @pl.when(s == 0)
    def _build():
        e0 = gi_ref[b, 0]
        e1 = gi_ref[b, 1]
        g0 = gv_ref[b, 0]
        g1 = gv_ref[b, 1]
        w1a = w1_ref[e0]
        w1b = w1_ref[e1]
        w2a = w2_ref[e0] * g0
        w2b = w2_ref[e1] * g1
        for i in range(G):
            o = i * HD
            wa_ref[pl.ds(o, HD), :] = jnp.zeros((HD, HW), jnp.float32)
            wa_ref[pl.ds(o, HD), pl.ds(o, HD)] = w1a
            wa_ref[pl.ds(o, HD), pl.ds(GW + o, HD)] = w1b
            wb_ref[pl.ds(o, HD), :] = jnp.zeros((HD, GW), jnp.float32)
            wb_ref[pl.ds(GW + o, HD), :] = jnp.zeros((HD, GW), jnp.float32)
            wb_ref[pl.ds(o, HD), pl.ds(o, HD)] = w2a
            wb_ref[pl.ds(GW + o, HD), pl.ds(o, HD)] = w2b
            b1c_ref[0, pl.ds(o, HD)] = b1_ref[e0]
            b1c_ref[0, pl.ds(GW + o, HD)] = b1_ref[e1]
            b2c_ref[0, pl.ds(o, HD)] = b2_ref[e0] * g0 + b2_ref[e1] * g1

    xt = x_ref[0]
    ngrp = xt.shape[1] // GW
    for gidx in range(ngrp):
        xg = xt[:, gidx * GW:(gidx + 1) * GW]
        h = jnp.dot(xg, wa_ref[...], preferred_element_type=jnp.float32)
        h = _gelu(h + b1c_ref[...])
        og = jnp.dot(h, wb_ref[...], preferred_element_type=jnp.float32)
        y_ref[:, gidx * GW:(gidx + 1) * GW] = xg + og + b2c_ref[...]
    out_ref[0] = (jnp.dot(y_ref[...], w3_ref[...],
                          preferred_element_type=jnp.float32) + b3_ref[...])


def kernel(x, W_sw, b_sw, W1, b1, W2, b2, W3, b3):
    B, S, D = x.shape
    E = W_sw.shape[1]
    HD = W1.shape[1]
    HIDDEN, _ = W3.shape

    CHUNK = 8192
    NL = W_sw.shape[0]
    nsteps = NL // CHUNK
    x0 = x.reshape(B, S * D)
    gv, gi = pl.pallas_call(
        functools.partial(_router_body, nsteps),
        grid=(nsteps,),
        in_specs=[
            pl.BlockSpec((B, CHUNK), lambda g: (0, g)),
            pl.BlockSpec((CHUNK, E), lambda g: (g, 0)),
            pl.BlockSpec((1, E), lambda g: (0, 0)),
        ],
        out_specs=[
            pl.BlockSpec((B, E), lambda g: (0, 0)),
            pl.BlockSpec((B, E), lambda g: (0, 0)),
        ],
        out_shape=[
            jax.ShapeDtypeStruct((B, E), jnp.float32),
            jax.ShapeDtypeStruct((B, E), jnp.int32),
        ],
        scratch_shapes=[pltpu.VMEM((B, E), jnp.float32)],
    )(x0, W_sw, b_sw.reshape(1, E))

    T = 256
    G = 4
    out = pl.pallas_call(
        functools.partial(_moe_body, S, T, HD, G),
        grid=(B, S // T),
        in_specs=[
            pl.BlockSpec((1, T, D), lambda b, s: (b, s, 0)),
            pl.BlockSpec(memory_space=pltpu.SMEM),
            pl.BlockSpec(memory_space=pltpu.SMEM),
            pl.BlockSpec(W1.shape, lambda b, s: (0, 0, 0)),
            pl.BlockSpec(b1.shape, lambda b, s: (0, 0)),
            pl.BlockSpec(W2.shape, lambda b, s: (0, 0, 0)),
            pl.BlockSpec(b2.shape, lambda b, s: (0, 0)),
            pl.BlockSpec(W3.shape, lambda b, s: (0, 0)),
            pl.BlockSpec((1, D), lambda b, s: (0, 0)),
        ],
        out_specs=pl.BlockSpec((1, T, D), lambda b, s: (b, s, 0)),
        out_shape=jax.ShapeDtypeStruct((B, S, D), jnp.float32),
        scratch_shapes=[
            pltpu.VMEM((G * HD, 2 * G * HD), jnp.float32),
            pltpu.VMEM((2 * G * HD, G * HD), jnp.float32),
            pltpu.VMEM((1, 2 * G * HD), jnp.float32),
            pltpu.VMEM((1, G * HD), jnp.float32),
            pltpu.VMEM((T, D), jnp.float32),
        ],
    )(x, gv, gi, W1, b1, W2, b2, W3, b3.reshape(1, D))
    return out
